# baseline (speedup 1.0000x reference)
import functools

import jax
import jax.numpy as jnp
import numpy as np
from jax import lax
from jax.experimental import pallas as pl
from jax.experimental.pallas import tpu as pltpu
from jax.experimental.pallas import tpu_sc as plsc

L = 16


def _build_kernel(B, J, C, K, NW, CH, JS):
    KG = -(-K // L)
    NB = CH // L
    rows_per_w = B // NW
    nchunks = rows_per_w // CH
    nsteps = nchunks * C
    mesh = plsc.VectorSubcoreMesh(core_axis_name="c", subcore_axis_name="s")
    NC = mesh.num_cores

    @functools.partial(
        pl.kernel,
        out_type=jax.ShapeDtypeStruct((C, K, B), jnp.float32),
        mesh=mesh,
        compiler_params=pltpu.CompilerParams(needs_layout_passes=False),
        scratch_types=[
            pltpu.VMEM((K,), jnp.int32),
            pltpu.VMEM((KG * L * L,), jnp.int32),
            pltpu.VMEM((CH, JS), jnp.float32),
            pltpu.VMEM((CH, JS), jnp.float32),
            pltpu.VMEM((K, CH), jnp.float32),
            pltpu.VMEM((K, CH), jnp.float32),
            pltpu.SemaphoreType.DMA,
            pltpu.SemaphoreType.DMA,
            pltpu.SemaphoreType.DMA,
            pltpu.SemaphoreType.DMA,
        ],
    )
    def _k(jt_hbm, jm_hbm, ktab_hbm, out_hbm, jm_v, ktab_v,
           in_a, in_b, out_a, out_b, isem_a, isem_b, osem_a, osem_b):
        wid = lax.axis_index("s") * NC + lax.axis_index("c")
        stage_jm = pltpu.make_async_copy(jm_hbm, jm_v, osem_a)
        stage_kt = pltpu.make_async_copy(ktab_hbm, ktab_v, osem_b)
        stage_jm.start()
        stage_kt.start()
        iota = lax.iota(jnp.int32, L)
        base = wid * rows_per_w

        ins = [in_a, in_b]
        outs = [out_a, out_b]
        isems = [isem_a, isem_b]
        osems = [osem_a, osem_b]

        def in_copy(p):
            cc, c = divmod(p, C)
            src = jt_hbm.at[c].at[pl.ds(base + cc * CH, CH), pl.ds(0, JS)]
            return pltpu.make_async_copy(src, ins[p % 2], isems[p % 2])

        kranges = [(kg * L, min(L, K - kg * L)) for kg in range(KG)]

        def out_copy(p, kg):
            cc, c = divmod(p, C)
            r0, rn = kranges[kg]
            dst = out_hbm.at[c].at[pl.ds(r0, rn), pl.ds(base + cc * CH, CH)]
            return pltpu.make_async_copy(
                outs[p % 2].at[pl.ds(r0, rn)], dst, osems[p % 2])

        in_copy(0).start()
        stage_jm.wait()
        stage_kt.wait()
        for p in range(nsteps):
            if p + 1 < nsteps:
                in_copy(p + 1).start()
            in_copy(p).wait()
            if p >= 2:
                for kg in range(KG):
                    out_copy(p - 2, kg).wait()
            in_v, out_v = ins[p % 2], outs[p % 2]

            for kg in range(KG):
                kvs = [ktab_v[pl.ds((kg * L + d) * L, L)] for d in range(L)]
                jvs = [plsc.load_gather(jm_v, [kv]) for kv in kvs]

                def body(i, carry):
                    bvec0 = (2 * i) * L + iota
                    bvec1 = bvec0 + L
                    vals0 = [plsc.load_gather(in_v, [bvec0, jv])
                             for jv in jvs]
                    vals1 = []
                    for d in range(L):
                        vals1.append(plsc.load_gather(in_v, [bvec1, jvs[d]]))
                        plsc.store_scatter(out_v, [kvs[d], bvec0], vals0[d])
                    for d in range(L):
                        plsc.store_scatter(out_v, [kvs[d], bvec1], vals1[d])
                    return carry

                lax.fori_loop(0, NB // 2, body, 0)
                out_copy(p, kg).start()
        for p in (nsteps - 2, nsteps - 1):
            for kg in range(KG):
                out_copy(p, kg).wait()

    def run(joints, joint_maps):
        lanes = np.arange(L)
        ktab = np.empty((KG, L, L), dtype=np.int32)
        for kg in range(KG):
            for d in range(L):
                ktab[kg, d] = np.minimum(kg * L + (lanes + d) % L, K - 1)
        jt = jnp.transpose(joints, (2, 0, 1))
        ot = _k(jt, joint_maps, jnp.asarray(ktab.reshape(-1)))
        return jnp.transpose(ot, (2, 1, 0))

    return run


def kernel(joints, joint_maps):
    B, J, C = joints.shape
    (K,) = joint_maps.shape
    run = _build_kernel(B, J, C, K, NW=32, CH=256, JS=J)
    return run(joints, joint_maps)


# speedup vs baseline: 1.0498x; 1.0498x over previous
import functools

import jax
import jax.numpy as jnp
import numpy as np
from jax import lax
from jax.experimental import pallas as pl
from jax.experimental.pallas import tpu as pltpu
from jax.experimental.pallas import tpu_sc as plsc

L = 16


def _build_kernel(B, J, C, K, NW, CH, JS):
    KG = -(-K // L)
    NB = CH // L
    rows_per_w = B // NW
    nchunks = rows_per_w // CH
    nsteps = nchunks * C
    mesh = plsc.VectorSubcoreMesh(core_axis_name="c", subcore_axis_name="s")
    NC = mesh.num_cores

    @functools.partial(
        pl.kernel,
        out_type=jax.ShapeDtypeStruct((C, K, B), jnp.float32),
        mesh=mesh,
        compiler_params=pltpu.CompilerParams(needs_layout_passes=False),
        scratch_types=[
            pltpu.VMEM((K,), jnp.int32),
            pltpu.VMEM((KG * L * L,), jnp.int32),
            pltpu.VMEM((CH, JS), jnp.float32),
            pltpu.VMEM((CH, JS), jnp.float32),
            pltpu.VMEM((K, CH), jnp.float32),
            pltpu.VMEM((K, CH), jnp.float32),
            pltpu.SemaphoreType.DMA,
            pltpu.SemaphoreType.DMA,
            pltpu.SemaphoreType.DMA,
            pltpu.SemaphoreType.DMA,
        ],
    )
    def _k(jt_hbm, jm_hbm, ktab_hbm, out_hbm, jm_v, ktab_v,
           in_a, in_b, out_a, out_b, isem_a, isem_b, osem_a, osem_b):
        wid = lax.axis_index("s") * NC + lax.axis_index("c")
        stage_jm = pltpu.make_async_copy(jm_hbm, jm_v, osem_a)
        stage_kt = pltpu.make_async_copy(ktab_hbm, ktab_v, osem_b)
        stage_jm.start()
        stage_kt.start()
        iota = lax.iota(jnp.int32, L)
        base = wid * rows_per_w

        ins = [in_a, in_b]
        outs = [out_a, out_b]
        isems = [isem_a, isem_b]
        osems = [osem_a, osem_b]

        def in_copy(p):
            cc, c = divmod(p, C)
            src = jt_hbm.at[c].at[pl.ds(base + cc * CH, CH), pl.ds(0, JS)]
            return pltpu.make_async_copy(src, ins[p % 2], isems[p % 2])

        kranges = [(kg * L, min(L, K - kg * L)) for kg in range(KG)]

        def out_copy(p, kg):
            cc, c = divmod(p, C)
            r0, rn = kranges[kg]
            dst = out_hbm.at[c].at[pl.ds(r0, rn), pl.ds(base + cc * CH, CH)]
            return pltpu.make_async_copy(
                outs[p % 2].at[pl.ds(r0, rn)], dst, osems[p % 2])

        in_copy(0).start()
        stage_jm.wait()
        stage_kt.wait()
        for p in range(nsteps):
            if p + 1 < nsteps:
                in_copy(p + 1).start()
            in_copy(p).wait()
            if p >= 2:
                for kg in range(KG):
                    out_copy(p - 2, kg).wait()
            in_v, out_v = ins[p % 2], outs[p % 2]

            for kg in range(KG):
                kvs = [ktab_v[pl.ds((kg * L + d) * L, L)] for d in range(L)]
                jvs = [plsc.load_gather(jm_v, [kv]) for kv in kvs]

                def body(bg, carry):
                    bvec = bg * L + iota
                    vals = [plsc.load_gather(in_v, [bvec, jv]) for jv in jvs]
                    for d in range(L):
                        plsc.store_scatter(out_v, [kvs[d], bvec], vals[d])
                    return carry

                lax.fori_loop(0, NB, body, 0)
                out_copy(p, kg).start()
        for p in (nsteps - 2, nsteps - 1):
            for kg in range(KG):
                out_copy(p, kg).wait()

    def run(joints, joint_maps):
        lanes = np.arange(L)
        ktab = np.empty((KG, L, L), dtype=np.int32)
        for kg in range(KG):
            for d in range(L):
                ktab[kg, d] = np.minimum(kg * L + (lanes + d) % L, K - 1)
        jt = jnp.transpose(joints, (2, 0, 1))
        ot = _k(jt, joint_maps, jnp.asarray(ktab.reshape(-1)))
        return jnp.transpose(ot, (2, 1, 0))

    return run


def kernel(joints, joint_maps):
    B, J, C = joints.shape
    (K,) = joint_maps.shape
    run = _build_kernel(B, J, C, K, NW=32, CH=256, JS=J)
    return run(joints, joint_maps)


# speedup vs baseline: 1.0553x; 1.0053x over previous
import functools

import jax
import jax.numpy as jnp
import numpy as np
from jax import lax
from jax.experimental import pallas as pl
from jax.experimental.pallas import tpu as pltpu
from jax.experimental.pallas import tpu_sc as plsc

L = 16


def _build_kernel(B, J, C, K, NW, CH):
    KG = -(-K // L)
    NB = CH // L
    rows_per_w = B // NW
    nchunks = rows_per_w // CH
    nsteps = nchunks * C
    mesh = plsc.VectorSubcoreMesh(core_axis_name="c", subcore_axis_name="s")
    NC = mesh.num_cores

    @functools.partial(
        pl.kernel,
        out_type=jax.ShapeDtypeStruct((C, K, B), jnp.float32),
        mesh=mesh,
        compiler_params=pltpu.CompilerParams(needs_layout_passes=False),
        scratch_types=[
            pltpu.VMEM((K,), jnp.int32),
            pltpu.VMEM((KG * L * L,), jnp.int32),
            pltpu.VMEM((CH, J), jnp.float32),
            pltpu.VMEM((CH, J), jnp.float32),
            pltpu.VMEM((K, CH), jnp.float32),
            pltpu.VMEM((K, CH), jnp.float32),
            pltpu.SemaphoreType.DMA,
            pltpu.SemaphoreType.DMA,
            pltpu.SemaphoreType.DMA,
            pltpu.SemaphoreType.DMA,
        ],
    )
    def _k(jt_hbm, jm_hbm, ktab_hbm, out_hbm, jm_v, ktab_v,
           in_a, in_b, out_a, out_b, isem_a, isem_b, osem_a, osem_b):
        wid = lax.axis_index("s") * NC + lax.axis_index("c")
        stage_jm = pltpu.make_async_copy(jm_hbm, jm_v, osem_a)
        stage_kt = pltpu.make_async_copy(ktab_hbm, ktab_v, osem_b)
        stage_jm.start()
        stage_kt.start()
        iota = lax.iota(jnp.int32, L)
        base = wid * rows_per_w

        ins = [in_a, in_b]
        outs = [out_a, out_b]
        isems = [isem_a, isem_b]
        osems = [osem_a, osem_b]

        def in_copy(p):
            cc, c = divmod(p, C)
            src = jt_hbm.at[c].at[pl.ds(base + cc * CH, CH)]
            return pltpu.make_async_copy(src, ins[p % 2], isems[p % 2])

        kranges = [(kg * L, min(L, K - kg * L)) for kg in range(KG)]

        def out_copy(p, kg):
            cc, c = divmod(p, C)
            r0, rn = kranges[kg]
            dst = out_hbm.at[c].at[pl.ds(r0, rn), pl.ds(base + cc * CH, CH)]
            return pltpu.make_async_copy(
                outs[p % 2].at[pl.ds(r0, rn)], dst, osems[p % 2])

        in_copy(0).start()
        stage_jm.wait()
        stage_kt.wait()
        for p in range(nsteps):
            if p + 1 < nsteps:
                in_copy(p + 1).start()
            in_copy(p).wait()
            if p >= 2:
                for kg in range(KG):
                    out_copy(p - 2, kg).wait()
            in_v, out_v = ins[p % 2], outs[p % 2]

            for kg in range(KG):
                kvs = [ktab_v[pl.ds((kg * L + d) * L, L)] for d in range(L)]
                jvs = [plsc.load_gather(jm_v, [kv]) for kv in kvs]

                def body(bg, carry):
                    bvec = bg * L + iota
                    vals = [plsc.load_gather(in_v, [bvec, jv]) for jv in jvs]
                    for d in range(L):
                        plsc.store_scatter(out_v, [kvs[d], bvec], vals[d])
                    return carry

                lax.fori_loop(0, NB, body, 0)
                out_copy(p, kg).start()
        for p in (nsteps - 2, nsteps - 1):
            for kg in range(KG):
                out_copy(p, kg).wait()

    def run(joints, joint_maps):
        lanes = np.arange(L)
        ktab = np.empty((KG, L, L), dtype=np.int32)
        for kg in range(KG):
            for d in range(L):
                ktab[kg, d] = np.minimum(kg * L + (lanes + d) % L, K - 1)
        jt = jnp.transpose(joints, (2, 0, 1))
        ot = _k(jt, joint_maps, jnp.asarray(ktab.reshape(-1)))
        return jnp.transpose(ot, (2, 1, 0))

    return run


def kernel(joints, joint_maps):
    B, J, C = joints.shape
    (K,) = joint_maps.shape
    run = _build_kernel(B, J, C, K, NW=32, CH=256)
    return run(joints, joint_maps)
